# baseline (speedup 1.0000x reference)
import functools
import jax
import jax.numpy as jnp
from jax import lax
from jax.experimental import pallas as pl
from jax.experimental.pallas import tpu as pltpu
from jax.experimental.pallas import tpu_sc as plsc

_CHUNK = 16 * 1024


def _sc_body(x_hbm, pos_hbm, out_hbm, xv0, xv1, pv0, pv1,
             sx0, sx1, sp0, sp1, so0, so1, nc, nw, s_elems):
    wid = lax.axis_index("s") * nc + lax.axis_index("c")
    total = x_hbm.shape[0]
    per_w = total // nw
    base = wid * per_w
    pos_base = lax.rem(base, s_elems)
    n_chunks = per_w // _CHUNK

    xv = (xv0, xv1)
    pv = (pv0, pv1)
    sx = (sx0, sx1)
    sp = (sp0, sp1)
    so = (so0, so1)

    def start_in(c):
        b = c % 2
        dx = pltpu.async_copy(
            x_hbm.at[pl.ds(base + c * _CHUNK, _CHUNK)], xv[b], sx[b])
        dp = pltpu.async_copy(
            pos_hbm.at[pl.ds(pos_base + c * _CHUNK, _CHUNK)], pv[b], sp[b])
        return dx, dp

    descs_in = [None] * n_chunks
    descs_out = [None] * n_chunks
    descs_in[0] = start_in(0)

    for c in range(n_chunks):
        b = c % 2
        dx, dp = descs_in[c]
        dx.wait()
        dp.wait()
        if c >= 1:
            descs_out[c - 1].wait()
        if c + 1 < n_chunks:
            descs_in[c + 1] = start_in(c + 1)

        xb, pb = xv[b], pv[b]

        @plsc.parallel_loop(0, _CHUNK // 16, 1, unroll=16)
        def _vec(k):
            col = pl.multiple_of(lax.shift_left(k, 4), 16)
            plsc.addupdate(xb.at[pl.ds(col, 16)], pb[pl.ds(col, 16)])

        descs_out[c] = pltpu.async_copy(
            xb, out_hbm.at[pl.ds(base + c * _CHUNK, _CHUNK)], so[b])

    descs_out[n_chunks - 1].wait()


def kernel(x, pos_table):
    B, S, D = x.shape
    pos = pos_table[:S].reshape(S * D)
    x1 = x.reshape(B * S * D)
    info = plsc.get_sparse_core_info()
    nc, ns = info.num_cores, info.num_subcores
    nw = nc * ns
    mesh = plsc.VectorSubcoreMesh(core_axis_name="c", subcore_axis_name="s")
    body = functools.partial(_sc_body, nc=nc, nw=nw, s_elems=S * D)
    run = pl.kernel(
        body,
        out_type=jax.ShapeDtypeStruct((B * S * D,), x.dtype),
        mesh=mesh,
        scratch_types=[
            pltpu.VMEM((_CHUNK,), jnp.float32),
            pltpu.VMEM((_CHUNK,), jnp.float32),
            pltpu.VMEM((_CHUNK,), jnp.float32),
            pltpu.VMEM((_CHUNK,), jnp.float32),
            pltpu.SemaphoreType.DMA,
            pltpu.SemaphoreType.DMA,
            pltpu.SemaphoreType.DMA,
            pltpu.SemaphoreType.DMA,
            pltpu.SemaphoreType.DMA,
            pltpu.SemaphoreType.DMA,
        ],
    )
    return run(x1, pos).reshape(B, S, D)


# speedup vs baseline: 5.2767x; 5.2767x over previous
import jax
import jax.numpy as jnp
from jax.experimental import pallas as pl

_BLOCK = 1024


def _add_kernel(x_ref, pos_ref, o_ref):
    o_ref[...] = x_ref[...] + pos_ref[...]


def kernel(x, pos_table):
    B, S, D = x.shape
    pos = pos_table[:S]
    x2 = x.reshape(B * S, D)
    sb = S // _BLOCK
    out = pl.pallas_call(
        _add_kernel,
        grid=(sb, B),
        in_specs=[
            pl.BlockSpec((_BLOCK, D), lambda s, b, sb=sb: (b * sb + s, 0)),
            pl.BlockSpec((_BLOCK, D), lambda s, b: (s, 0)),
        ],
        out_specs=pl.BlockSpec((_BLOCK, D), lambda s, b, sb=sb: (b * sb + s, 0)),
        out_shape=jax.ShapeDtypeStruct((B * S, D), x.dtype),
    )(x2, pos)
    return out.reshape(B, S, D)


# speedup vs baseline: 5.7020x; 1.0806x over previous
import jax
import jax.numpy as jnp
from jax.experimental import pallas as pl

_BLOCK = 2048


def _add_kernel(x_ref, pos_ref, o_ref):
    o_ref[...] = x_ref[...] + pos_ref[...]


def kernel(x, pos_table):
    B, S, D = x.shape
    pos = pos_table[:S]
    x2 = x.reshape(B * S, D)
    sb = S // _BLOCK
    out = pl.pallas_call(
        _add_kernel,
        grid=(sb, B),
        in_specs=[
            pl.BlockSpec((_BLOCK, D), lambda s, b, sb=sb: (b * sb + s, 0)),
            pl.BlockSpec((_BLOCK, D), lambda s, b: (s, 0)),
        ],
        out_specs=pl.BlockSpec((_BLOCK, D), lambda s, b, sb=sb: (b * sb + s, 0)),
        out_shape=jax.ShapeDtypeStruct((B * S, D), x.dtype),
    )(x2, pos)
    return out.reshape(B, S, D)
